# baseline (speedup 1.0000x reference)
import functools

import jax
import jax.numpy as jnp
from jax import lax
from jax.experimental import pallas as pl
from jax.experimental.pallas import tpu as pltpu
from jax.experimental.pallas import tpu_sc as plsc

_B, _T, _C = 16, 1024, 999
_N = _B * _T

_M = 8192
_OPS = 4
_ROWS = 512
_STEPS = _M // (_OPS * _ROWS)

_NC, _NS, _L = 2, 16, 16
_NW = _NC * _NS
_SCROWS = _N - _M
_PWR = _SCROWS // _NW
_KR = 32
_NCH = _PWR // _KR
_FULLV = (_C - _L) // _L + 1
_TAILC = _C - _L
_TAILN = _C - _FULLV * _L
assert _PWR % _KR == 0 and _FULLV == 62 and _TAILN == 7


def _tc_body(*refs):
    x_refs = refs[:_OPS]
    t_refs = refs[_OPS:2 * _OPS]
    o_ref = refs[2 * _OPS]
    col = lax.broadcasted_iota(jnp.int32, (_ROWS, _C), 1)
    part = jnp.float32(0.0)
    for x_ref, t_ref in zip(x_refs, t_refs):
        x = x_ref[...]
        t = t_ref[...]
        hit = col == (t - 1)
        part += jnp.sum(x * x) - 2.0 * jnp.sum(jnp.where(hit, x, 0.0))

    @pl.when(pl.program_id(0) == 0)
    def _():
        o_ref[0, 0] = 0.0

    o_ref[0, 0] += part


_sc_mesh = plsc.VectorSubcoreMesh(core_axis_name="c", subcore_axis_name="s")


@functools.partial(
    pl.kernel,
    mesh=_sc_mesh,
    out_type=jax.ShapeDtypeStruct((_NW, _L), jnp.float32),
    scratch_types=[
        pltpu.VMEM((_KR, _C), jnp.float32),
        pltpu.VMEM((_KR, _C), jnp.float32),
        pltpu.VMEM((_PWR,), jnp.int32),
        pltpu.VMEM((_L,), jnp.float32),
        pltpu.SemaphoreType.DMA,
        pltpu.SemaphoreType.DMA,
    ],
    compiler_params=pltpu.CompilerParams(needs_layout_passes=False),
)
def _sc_tail(x_hbm, t_hbm, out_hbm, buf0_v, buf1_v, t_v, acc_v, sem0, sem1):
    wid = lax.axis_index("s") * _NC + lax.axis_index("c")
    tail_row = wid * _PWR
    bidx = tail_row // _T
    brow = tail_row % _T
    rbase = _M + tail_row
    pltpu.sync_copy(t_hbm.at[pl.ds(rbase, _PWR)], t_v)

    bufs = [buf0_v, buf1_v]
    sems = [sem0, sem1]
    copies = [None, None]
    copies[0] = pltpu.async_copy(
        x_hbm.at[bidx, pl.ds(brow, _KR), :], bufs[0], sems[0])
    acc = jnp.zeros((_L,), jnp.float32)
    lane = lax.iota(jnp.int32, _L)
    for c in range(_NCH):
        b = c % 2
        if c + 1 < _NCH:
            copies[1 - b] = pltpu.async_copy(
                x_hbm.at[bidx, pl.ds(brow + (c + 1) * _KR, _KR), :],
                bufs[1 - b], sems[1 - b])
        copies[b].wait()
        buf = bufs[b]

        def col_step(j, a, buf=buf):
            for r in range(_KR):
                v = buf[r, pl.ds(j * _L, _L)]
                a = a + v * v
            return a

        acc = lax.fori_loop(0, _FULLV, col_step, acc)
        for r in range(_KR):
            v = buf[r, pl.ds(_TAILC, _L)]
            acc = acc + jnp.where(lane >= _L - _TAILN, v * v, 0.0)

        for g in range(_KR // _L):
            tt = t_v[pl.ds(c * _KR + g * _L, _L)]
            rows = g * _L + lane
            cols = jnp.maximum(tt - 1, 0)
            gv = plsc.load_gather(buf, [rows, cols])
            acc = acc - 2.0 * jnp.where(tt >= 1, gv, 0.0)

    acc_v[...] = acc
    pltpu.sync_copy(acc_v, out_hbm.at[wid])


def kernel(rel_ress, targets, mask):
    del mask
    x = rel_ress.reshape(_N, _C)
    t_flat = targets.astype(jnp.int32).reshape(_N)
    sc_parts = _sc_tail(rel_ress[_M // _T:], t_flat)

    t_col = t_flat.reshape(_N, 1)
    x_specs = [
        pl.BlockSpec((_ROWS, _C), lambda i, k=k: (i + k * _STEPS, 0))
        for k in range(_OPS)
    ]
    t_specs = [
        pl.BlockSpec((_ROWS, 1), lambda i, k=k: (i + k * _STEPS, 0))
        for k in range(_OPS)
    ]

    hsteps = _M // (2 * _OPS * _ROWS)

    def tc_half(row0_blocks):
        xs = [
            pl.BlockSpec(
                (_ROWS, _C),
                lambda i, k=k, r=row0_blocks: (r + i + k * hsteps, 0))
            for k in range(_OPS)
        ]
        ts = [
            pl.BlockSpec(
                (_ROWS, 1),
                lambda i, k=k, r=row0_blocks: (r + i + k * hsteps, 0))
            for k in range(_OPS)
        ]
        return pl.pallas_call(
            _tc_body,
            grid=(hsteps,),
            in_specs=xs + ts,
            out_specs=pl.BlockSpec(memory_space=pltpu.SMEM),
            out_shape=jax.ShapeDtypeStruct((1, 1), jnp.float32),
        )(*([x] * _OPS + [t_col] * _OPS))

    half_blocks = _M // (2 * _ROWS)
    out_a = tc_half(0)
    out_b = tc_half(half_blocks)
    total = out_a[0, 0] + out_b[0, 0] + jnp.sum(sc_parts)
    return (total + jnp.float32(_N)) / jnp.float32(_N * (_C + 1))


# speedup vs baseline: 1.2496x; 1.2496x over previous
import functools

import jax
import jax.numpy as jnp
from jax import lax
from jax.experimental import pallas as pl
from jax.experimental.pallas import tpu as pltpu
from jax.experimental.pallas import tpu_sc as plsc

_B, _T, _C = 16, 1024, 999
_N = _B * _T

_M = 12288
_OPS = 4
_ROWS = 512
_STEPS = _M // (_OPS * _ROWS)

_NC, _NS, _L = 2, 16, 16
_NW = _NC * _NS
_SCROWS = _N - _M
_PWR = _SCROWS // _NW
_KR = 32
_NCH = _PWR // _KR
_FULLV = (_C - _L) // _L + 1
_TAILC = _C - _L
_TAILN = _C - _FULLV * _L
assert _PWR % _KR == 0 and _FULLV == 62 and _TAILN == 7


def _tc_body(*refs):
    x_refs = refs[:_OPS]
    t_refs = refs[_OPS:2 * _OPS]
    o_ref = refs[2 * _OPS]
    col = lax.broadcasted_iota(jnp.int32, (_ROWS, _C), 1)
    part = jnp.float32(0.0)
    for x_ref, t_ref in zip(x_refs, t_refs):
        x = x_ref[...]
        t = t_ref[...]
        hit = col == (t - 1)
        part += jnp.sum(x * x) - 2.0 * jnp.sum(jnp.where(hit, x, 0.0))

    @pl.when(pl.program_id(0) == 0)
    def _():
        o_ref[0, 0] = 0.0

    o_ref[0, 0] += part


_sc_mesh = plsc.VectorSubcoreMesh(core_axis_name="c", subcore_axis_name="s")


@functools.partial(
    pl.kernel,
    mesh=_sc_mesh,
    out_type=jax.ShapeDtypeStruct((_NW, _L), jnp.float32),
    scratch_types=[
        pltpu.VMEM((_KR, _C), jnp.float32),
        pltpu.VMEM((_KR, _C), jnp.float32),
        pltpu.VMEM((_PWR,), jnp.int32),
        pltpu.VMEM((_L,), jnp.float32),
        pltpu.SemaphoreType.DMA,
        pltpu.SemaphoreType.DMA,
    ],
    compiler_params=pltpu.CompilerParams(needs_layout_passes=False),
)
def _sc_tail(x_hbm, t_hbm, out_hbm, buf0_v, buf1_v, t_v, acc_v, sem0, sem1):
    wid = lax.axis_index("s") * _NC + lax.axis_index("c")
    rbase = _M + wid * _PWR
    pltpu.sync_copy(t_hbm.at[pl.ds(rbase, _PWR)], t_v)

    bufs = [buf0_v, buf1_v]
    sems = [sem0, sem1]
    copies = [None, None]
    copies[0] = pltpu.async_copy(
        x_hbm.at[pl.ds(rbase, _KR), :], bufs[0], sems[0])
    acc = jnp.zeros((_L,), jnp.float32)
    lane = lax.iota(jnp.int32, _L)
    for c in range(_NCH):
        b = c % 2
        if c + 1 < _NCH:
            copies[1 - b] = pltpu.async_copy(
                x_hbm.at[pl.ds(rbase + (c + 1) * _KR, _KR), :],
                bufs[1 - b], sems[1 - b])
        copies[b].wait()
        buf = bufs[b]

        def col_step(j, a, buf=buf):
            for r in range(_KR):
                v = buf[r, pl.ds(j * _L, _L)]
                a = a + v * v
            return a

        acc = lax.fori_loop(0, _FULLV, col_step, acc)
        for r in range(_KR):
            v = buf[r, pl.ds(_TAILC, _L)]
            acc = acc + jnp.where(lane >= _L - _TAILN, v * v, 0.0)

        for g in range(_KR // _L):
            tt = t_v[pl.ds(c * _KR + g * _L, _L)]
            rows = g * _L + lane
            cols = jnp.maximum(tt - 1, 0)
            gv = plsc.load_gather(buf, [rows, cols])
            acc = acc - 2.0 * jnp.where(tt >= 1, gv, 0.0)

    acc_v[...] = acc
    pltpu.sync_copy(acc_v, out_hbm.at[wid])


def kernel(rel_ress, targets, mask):
    del mask
    x = rel_ress.reshape(_N, _C)
    t_flat = targets.astype(jnp.int32).reshape(_N)
    sc_parts = _sc_tail(x, t_flat)

    t_col = t_flat.reshape(_N, 1)
    x_specs = [
        pl.BlockSpec((_ROWS, _C), lambda i, k=k: (i + k * _STEPS, 0))
        for k in range(_OPS)
    ]
    t_specs = [
        pl.BlockSpec((_ROWS, 1), lambda i, k=k: (i + k * _STEPS, 0))
        for k in range(_OPS)
    ]
    out = pl.pallas_call(
        _tc_body,
        grid=(_STEPS,),
        in_specs=x_specs + t_specs,
        out_specs=pl.BlockSpec(memory_space=pltpu.SMEM),
        out_shape=jax.ShapeDtypeStruct((1, 1), jnp.float32),
    )(*([x] * _OPS + [t_col] * _OPS))
    total = out[0, 0] + jnp.sum(sc_parts)
    return (total + jnp.float32(_N)) / jnp.float32(_N * (_C + 1))
